# baseline (speedup 1.0000x reference)
import jax
import jax.numpy as jnp
from jax.experimental import pallas as pl
from jax.experimental.pallas import tpu as pltpu

PROJ_WIDTH = 128
BASE = 10000.0
BS = 4096
H_BLK = 4


def _rope_body(sid_ref, perm_ref, ecos_ref, esin_ref, x_ref, o_ref,
               cos_ref, sin_ref):
    h = pl.program_id(2)

    @pl.when(h == 0)
    def _compute_trig():
        half = PROJ_WIDTH // 2
        sid = sid_ref[0, 0, :].astype(jnp.float32)
        k = jax.lax.broadcasted_iota(jnp.int32, (BS, half), 1)
        expnt = k.astype(jnp.float32) * (2.0 / PROJ_WIDTH)
        theta = jnp.exp(-jnp.log(BASE) * expnt)
        angle = sid[:, None] * theta
        c64 = jnp.cos(angle)
        s64 = jnp.sin(angle)
        cos_ref[...] = jnp.dot(c64, ecos_ref[...],
                               preferred_element_type=jnp.float32,
                               precision=jax.lax.Precision.HIGHEST)
        sin_ref[...] = jnp.dot(s64, esin_ref[...],
                               preferred_element_type=jnp.float32,
                               precision=jax.lax.Precision.HIGHEST)

    c = cos_ref[...]
    s = sin_ref[...]
    p = perm_ref[...]
    for i in range(H_BLK):
        xi = x_ref[0, i]
        swp = jnp.dot(xi, p, preferred_element_type=jnp.float32)
        o_ref[0, i] = c * xi + s * swp


@jax.jit
def kernel(x, seq_id):
    B, H1, H2, S, W = x.shape
    H = H1 * H2
    n_sblk = S // BS
    xr = x.reshape(B, H, S, W)
    sid = seq_id.reshape(B * n_sblk, 1, BS)
    j = jnp.arange(W)
    perm = (j[:, None] ^ 1 == j[None, :]).astype(jnp.float32)
    k = jnp.arange(W // 2)
    pair = k[:, None] == j[None, :] // 2
    ecos = pair.astype(jnp.float32)
    esin = pair.astype(jnp.float32) * jnp.where(j % 2 == 0, -1.0, 1.0)

    out = pl.pallas_call(
        _rope_body,
        grid=(B, n_sblk, H // H_BLK),
        in_specs=[
            pl.BlockSpec((1, 1, BS), lambda b, sblk, h: (b * n_sblk + sblk, 0, 0)),
            pl.BlockSpec((W, W), lambda b, sblk, h: (0, 0)),
            pl.BlockSpec((W // 2, W), lambda b, sblk, h: (0, 0)),
            pl.BlockSpec((W // 2, W), lambda b, sblk, h: (0, 0)),
            pl.BlockSpec((1, H_BLK, BS, W), lambda b, sblk, h: (b, h, sblk, 0)),
        ],
        out_specs=pl.BlockSpec((1, H_BLK, BS, W), lambda b, sblk, h: (b, h, sblk, 0)),
        out_shape=jax.ShapeDtypeStruct((B, H, S, W), jnp.float32),
        scratch_shapes=[
            pltpu.VMEM((BS, W), jnp.float32),
            pltpu.VMEM((BS, W), jnp.float32),
        ],
        compiler_params=pltpu.CompilerParams(
            vmem_limit_bytes=63 * 1024 * 1024,
        ),
    )(sid, perm, ecos, esin, xr)
    return out.reshape(B, H1, H2, S, W)


# speedup vs baseline: 1.0694x; 1.0694x over previous
import jax
import jax.numpy as jnp
from jax.experimental import pallas as pl
from jax.experimental.pallas import tpu as pltpu

PROJ_WIDTH = 128
BASE = 10000.0
BS = 4096
H_BLK = 4


def _rope_body(sid_ref, perm_ref, ecos_ref, esin_ref, x_ref, o_ref,
               cos_ref, sin_ref):
    h = pl.program_id(2)

    @pl.when(h == 0)
    def _compute_trig():
        half = PROJ_WIDTH // 2
        sid = sid_ref[0, 0, :].astype(jnp.float32)
        k = jax.lax.broadcasted_iota(jnp.int32, (BS, half), 1)
        expnt = k.astype(jnp.float32) * (2.0 / PROJ_WIDTH)
        theta = jnp.exp(-jnp.log(BASE) * expnt)
        angle = sid[:, None] * theta
        c64 = jnp.cos(angle)
        s64 = jnp.sin(angle)
        cos_ref[...] = jnp.dot(c64, ecos_ref[...],
                               preferred_element_type=jnp.float32)
        sin_ref[...] = jnp.dot(s64, esin_ref[...],
                               preferred_element_type=jnp.float32)

    c = cos_ref[...]
    s = sin_ref[...]
    p = perm_ref[...]
    for i in range(H_BLK):
        xi = x_ref[0, i]
        swp = jnp.dot(xi, p, preferred_element_type=jnp.float32)
        o_ref[0, i] = c * xi + s * swp


@jax.jit
def kernel(x, seq_id):
    B, H1, H2, S, W = x.shape
    H = H1 * H2
    n_sblk = S // BS
    xr = x.reshape(B, H, S, W)
    sid = seq_id.reshape(B * n_sblk, 1, BS)
    j = jnp.arange(W)
    perm = (j[:, None] ^ 1 == j[None, :]).astype(jnp.float32)
    k = jnp.arange(W // 2)
    pair = k[:, None] == j[None, :] // 2
    ecos = pair.astype(jnp.float32)
    esin = pair.astype(jnp.float32) * jnp.where(j % 2 == 0, -1.0, 1.0)

    out = pl.pallas_call(
        _rope_body,
        grid=(B, n_sblk, H // H_BLK),
        in_specs=[
            pl.BlockSpec((1, 1, BS), lambda b, sblk, h: (b * n_sblk + sblk, 0, 0)),
            pl.BlockSpec((W, W), lambda b, sblk, h: (0, 0)),
            pl.BlockSpec((W // 2, W), lambda b, sblk, h: (0, 0)),
            pl.BlockSpec((W // 2, W), lambda b, sblk, h: (0, 0)),
            pl.BlockSpec((1, H_BLK, BS, W), lambda b, sblk, h: (b, h, sblk, 0)),
        ],
        out_specs=pl.BlockSpec((1, H_BLK, BS, W), lambda b, sblk, h: (b, h, sblk, 0)),
        out_shape=jax.ShapeDtypeStruct((B, H, S, W), jnp.float32),
        scratch_shapes=[
            pltpu.VMEM((BS, W), jnp.float32),
            pltpu.VMEM((BS, W), jnp.float32),
        ],
        compiler_params=pltpu.CompilerParams(
            vmem_limit_bytes=63 * 1024 * 1024,
        ),
    )(sid, perm, ecos, esin, xr)
    return out.reshape(B, H1, H2, S, W)


# speedup vs baseline: 1.0709x; 1.0014x over previous
import jax
import jax.numpy as jnp
from jax.experimental import pallas as pl
from jax.experimental.pallas import tpu as pltpu

PROJ_WIDTH = 128
BASE = 10000.0
BS = 4096
H_BLK = 4


def _rope_body(sid_ref, perm_ref, ecos_ref, esin_ref, x_ref, o_ref,
               cos_ref, sin_ref):
    h = pl.program_id(2)

    @pl.when(h == 0)
    def _compute_trig():
        sid = sid_ref[0, 0, :].astype(jnp.float32)
        d = jax.lax.broadcasted_iota(jnp.int32, (BS, PROJ_WIDTH), 1)
        even = (d % 2) == 0
        expnt = ((d // 2) * 2).astype(jnp.float32) * (1.0 / PROJ_WIDTH)
        theta = jnp.exp(-jnp.log(BASE) * expnt)
        ofs = jnp.where(even, 0.0, 0.5 * jnp.pi)
        m = jnp.cos(sid[:, None] * theta - ofs)
        cos_ref[...] = jnp.where(even, m, pltpu.roll(m, 1, 1))
        sin_ref[...] = jnp.where(even, -pltpu.roll(m, PROJ_WIDTH - 1, 1), m)

    c = cos_ref[...]
    s = sin_ref[...]
    p = perm_ref[...]
    for i in range(H_BLK):
        xi = x_ref[0, i]
        swp = jnp.dot(xi, p, preferred_element_type=jnp.float32)
        o_ref[0, i] = c * xi + s * swp


@jax.jit
def kernel(x, seq_id):
    B, H1, H2, S, W = x.shape
    H = H1 * H2
    n_sblk = S // BS
    xr = x.reshape(B, H, S, W)
    sid = seq_id.reshape(B * n_sblk, 1, BS)
    j = jnp.arange(W)
    perm = (j[:, None] ^ 1 == j[None, :]).astype(jnp.float32)
    k = jnp.arange(W // 2)
    pair = k[:, None] == j[None, :] // 2
    ecos = pair.astype(jnp.float32)
    esin = pair.astype(jnp.float32) * jnp.where(j % 2 == 0, -1.0, 1.0)

    out = pl.pallas_call(
        _rope_body,
        grid=(B, n_sblk, H // H_BLK),
        in_specs=[
            pl.BlockSpec((1, 1, BS), lambda b, sblk, h: (b * n_sblk + sblk, 0, 0)),
            pl.BlockSpec((W, W), lambda b, sblk, h: (0, 0)),
            pl.BlockSpec((W // 2, W), lambda b, sblk, h: (0, 0)),
            pl.BlockSpec((W // 2, W), lambda b, sblk, h: (0, 0)),
            pl.BlockSpec((1, H_BLK, BS, W), lambda b, sblk, h: (b, h, sblk, 0)),
        ],
        out_specs=pl.BlockSpec((1, H_BLK, BS, W), lambda b, sblk, h: (b, h, sblk, 0)),
        out_shape=jax.ShapeDtypeStruct((B, H, S, W), jnp.float32),
        scratch_shapes=[
            pltpu.VMEM((BS, W), jnp.float32),
            pltpu.VMEM((BS, W), jnp.float32),
        ],
        compiler_params=pltpu.CompilerParams(
            vmem_limit_bytes=63 * 1024 * 1024,
        ),
    )(sid, perm, ecos, esin, xr)
    return out.reshape(B, H1, H2, S, W)
